# Initial kernel scaffold; baseline (speedup 1.0000x reference)
import jax
import jax.numpy as jnp
from jax.experimental import pallas as pl


def kernel(x, edge_index, edge_weight, W1, b1, W2, b2):
    raise NotImplementedError("write your pallas kernel here")


# speedup vs baseline: 8.2136x; 8.2136x over previous
import functools

import jax
import jax.numpy as jnp
from jax import lax
from jax.experimental import pallas as pl
from jax.experimental.pallas import tpu as pltpu
from jax.experimental.pallas import tpu_sc as plsc

N = 10000
D = 128
E = 320000

NC = 2
NS = 16
L = 16
NW = NC * NS

BLK = 128
EPT = 10112
NB = EPT // BLK
EPAD = EPT * NW
NP = 10240

_MESH = plsc.VectorSubcoreMesh(
    core_axis_name="c", subcore_axis_name="s", num_cores=NC, num_subcores=NS)
_SC_PARAMS = pltpu.CompilerParams(needs_layout_passes=False)


def _zero_rows(rows, nrows):
    def body(r, _):
        for j in range(D // L):
            rows[r, pl.ds(j * L, L)] = jnp.zeros((L,), jnp.float32)
        return 0
    lax.fori_loop(0, nrows, body, 0)


def _scale_rows(rows, normb):
    def grp(g, _):
        nv = normb[pl.ds(g * L, L)]
        e0 = g * L
        for r in range(L):
            nrm = nv[r]
            for j in range(D // L):
                rows[e0 + r, pl.ds(j * L, L)] = (
                    rows[e0 + r, pl.ds(j * L, L)] * nrm)
        return 0
    lax.fori_loop(0, BLK // L, grp, 0)


def _deg_body(dst_hbm, w_hbm, degp_hbm, deg_l, dstb, wb):
    c = lax.axis_index("c")
    s = lax.axis_index("s")
    wid = s * NC + c

    def zero(i, _):
        deg_l[pl.ds(i * L, L)] = jnp.zeros((L,), jnp.float32)
        return 0
    lax.fori_loop(0, NP // L, zero, 0)

    base = wid * EPT

    def blk(b, _):
        off = base + b * BLK
        pltpu.sync_copy(dst_hbm.at[pl.ds(off, BLK)], dstb)
        pltpu.sync_copy(w_hbm.at[pl.ds(off, BLK)], wb)

        def vec(i, _):
            idx = dstb[pl.ds(i * L, L)]
            val = wb[pl.ds(i * L, L)]
            plsc.addupdate_scatter(deg_l, [idx], val)
            return 0
        lax.fori_loop(0, BLK // L, vec, 0)
        return 0
    lax.fori_loop(0, NB, blk, 0)

    pltpu.sync_copy(deg_l, degp_hbm.at[wid])


_deg_kernel = functools.partial(
    pl.kernel,
    out_type=jax.ShapeDtypeStruct((NW, NP), jnp.float32),
    mesh=_MESH,
    compiler_params=_SC_PARAMS,
    scratch_types=[
        pltpu.VMEM((NP,), jnp.float32),
        pltpu.VMEM((BLK,), jnp.int32),
        pltpu.VMEM((BLK,), jnp.float32),
    ],
)(_deg_body)


def _agg1_body(src_hbm, dst_hbm, w_hbm, dinv_hbm, h_hbm,
               aggp_hbm, norm_hbm,
               acc_sh, dinv_l, srcb, dstb, wb, normb, rows, sem):
    c = lax.axis_index("c")
    s = lax.axis_index("s")
    wid = s * NC + c

    pltpu.sync_copy(dinv_hbm, dinv_l)

    _zero_rows(rows, BLK)
    for k in range(NP // NS // BLK):
        pltpu.sync_copy(rows, acc_sh.at[pl.ds(s * (NP // NS) + k * BLK, BLK)])
    plsc.subcore_barrier()

    base = wid * EPT

    def blk(b, _):
        off = base + b * BLK
        pltpu.sync_copy(src_hbm.at[pl.ds(off, BLK)], srcb)
        pltpu.sync_copy(dst_hbm.at[pl.ds(off, BLK)], dstb)
        pltpu.sync_copy(w_hbm.at[pl.ds(off, BLK)], wb)
        cp = pltpu.async_copy(h_hbm.at[srcb], rows, sem)

        def nv(i, _):
            sv = srcb[pl.ds(i * L, L)]
            dv = dstb[pl.ds(i * L, L)]
            nrm = (plsc.load_gather(dinv_l, [sv]) *
                   plsc.load_gather(dinv_l, [dv]) *
                   wb[pl.ds(i * L, L)])
            normb[pl.ds(i * L, L)] = nrm
            return 0
        lax.fori_loop(0, BLK // L, nv, 0)
        pltpu.sync_copy(normb, norm_hbm.at[pl.ds(off, BLK)])
        cp.wait()

        _scale_rows(rows, normb)

        pltpu.sync_copy(rows, acc_sh.at[dstb], add=True)
        return 0
    lax.fori_loop(0, NB, blk, 0)

    plsc.subcore_barrier()
    pltpu.sync_copy(acc_sh.at[pl.ds(s * (NP // NS), NP // NS)],
                    aggp_hbm.at[c, pl.ds(s * (NP // NS), NP // NS)])


_agg1_kernel = functools.partial(
    pl.kernel,
    out_type=(jax.ShapeDtypeStruct((NC, NP, D), jnp.float32),
              jax.ShapeDtypeStruct((EPAD,), jnp.float32)),
    mesh=_MESH,
    compiler_params=_SC_PARAMS,
    scratch_types=[
        pltpu.VMEM_SHARED((NP, D), jnp.float32),
        pltpu.VMEM((NP,), jnp.float32),
        pltpu.VMEM((BLK,), jnp.int32),
        pltpu.VMEM((BLK,), jnp.int32),
        pltpu.VMEM((BLK,), jnp.float32),
        pltpu.VMEM((BLK,), jnp.float32),
        pltpu.VMEM((BLK, D), jnp.float32),
        pltpu.SemaphoreType.DMA,
    ],
)(_agg1_body)


def _agg2_body(src_hbm, dst_hbm, norm_hbm, h_hbm,
               aggp_hbm,
               acc_sh, srcb, dstb, normb, rows, sem):
    c = lax.axis_index("c")
    s = lax.axis_index("s")
    wid = s * NC + c

    _zero_rows(rows, BLK)
    for k in range(NP // NS // BLK):
        pltpu.sync_copy(rows, acc_sh.at[pl.ds(s * (NP // NS) + k * BLK, BLK)])
    plsc.subcore_barrier()

    base = wid * EPT

    def blk(b, _):
        off = base + b * BLK
        pltpu.sync_copy(src_hbm.at[pl.ds(off, BLK)], srcb)
        pltpu.sync_copy(dst_hbm.at[pl.ds(off, BLK)], dstb)
        pltpu.sync_copy(norm_hbm.at[pl.ds(off, BLK)], normb)
        cp = pltpu.async_copy(h_hbm.at[srcb], rows, sem)
        cp.wait()

        _scale_rows(rows, normb)

        pltpu.sync_copy(rows, acc_sh.at[dstb], add=True)
        return 0
    lax.fori_loop(0, NB, blk, 0)

    plsc.subcore_barrier()
    pltpu.sync_copy(acc_sh.at[pl.ds(s * (NP // NS), NP // NS)],
                    aggp_hbm.at[c, pl.ds(s * (NP // NS), NP // NS)])


_agg2_kernel = functools.partial(
    pl.kernel,
    out_type=jax.ShapeDtypeStruct((NC, NP, D), jnp.float32),
    mesh=_MESH,
    compiler_params=_SC_PARAMS,
    scratch_types=[
        pltpu.VMEM_SHARED((NP, D), jnp.float32),
        pltpu.VMEM((BLK,), jnp.int32),
        pltpu.VMEM((BLK,), jnp.int32),
        pltpu.VMEM((BLK,), jnp.float32),
        pltpu.VMEM((BLK, D), jnp.float32),
        pltpu.SemaphoreType.DMA,
    ],
)(_agg2_body)


def _mm1_body(x_ref, w_ref, degp_ref, h_ref, dinv_ref, dinv2_ref):
    deg = jnp.sum(degp_ref[...], axis=0) + 1.0
    dinv = jnp.where(deg > 0, lax.rsqrt(deg), 0.0)
    dinv_ref[...] = dinv
    dinv2_ref[...] = dinv * dinv
    h_ref[...] = jnp.dot(x_ref[...], w_ref[...],
                         preferred_element_type=jnp.float32)


_mm1 = pl.pallas_call(
    _mm1_body,
    out_shape=(jax.ShapeDtypeStruct((N, D), jnp.float32),
               jax.ShapeDtypeStruct((NP // 128, 128), jnp.float32),
               jax.ShapeDtypeStruct((NP // 128, 128), jnp.float32)))


def _mid_body(aggp_ref, h_ref, dinv2_ref, b_ref, w_ref, h2_ref):
    z = (aggp_ref[0, :N] + aggp_ref[1, :N] + h_ref[...] * dinv2_ref[...]
         + b_ref[...])
    z = jnp.maximum(z, 0.0)
    h2_ref[...] = jnp.dot(z, w_ref[...], preferred_element_type=jnp.float32)


_mid = pl.pallas_call(
    _mid_body,
    out_shape=jax.ShapeDtypeStruct((N, D), jnp.float32))


def _out_body(aggp_ref, h_ref, dinv2_ref, b_ref, x_ref, out_ref):
    z = (aggp_ref[0, :N] + aggp_ref[1, :N] + h_ref[...] * dinv2_ref[...]
         + b_ref[...])
    out_ref[:, :D] = jnp.maximum(z, 0.0)
    out_ref[:, D:] = x_ref[...]


_out = pl.pallas_call(
    _out_body,
    out_shape=jax.ShapeDtypeStruct((N, 2 * D), jnp.float32))


def kernel(x, edge_index, edge_weight, W1, b1, W2, b2):
    src = edge_index[0].astype(jnp.int32)
    dst = edge_index[1].astype(jnp.int32)
    w = edge_weight.astype(jnp.float32)
    pad = EPAD - E
    src_p = jnp.pad(src, (0, pad))
    dst_p = jnp.pad(dst, (0, pad))
    w_p = jnp.pad(w, (0, pad))

    degp = _deg_kernel(dst_p, w_p)
    h1, dinv2d, dinv2_2d = _mm1(x, W1, degp.reshape(NW, NP // 128, 128))
    dinv = dinv2d.reshape(NP)
    dinv2c = dinv2_2d.reshape(NP)[:N].reshape(N, 1)

    aggp1, norm = _agg1_kernel(src_p, dst_p, w_p, dinv, h1)
    h2 = _mid(aggp1, h1, dinv2c, b1.reshape(1, D), W2)
    aggp2 = _agg2_kernel(src_p, dst_p, norm, h2)
    return _out(aggp2, h2, dinv2c, b2.reshape(1, D), x)


# speedup vs baseline: 8.5479x; 1.0407x over previous
import functools

import jax
import jax.numpy as jnp
from jax import lax
from jax.experimental import pallas as pl
from jax.experimental.pallas import tpu as pltpu
from jax.experimental.pallas import tpu_sc as plsc

N = 10000
D = 128
E = 320000

NC = 2
NS = 16
L = 16
NW = NC * NS

BLK = 128
NB = 80
EPT = NB * BLK
EPAD = EPT * NW
NP = 10240
RPT = NP // NS

_MESH = plsc.VectorSubcoreMesh(
    core_axis_name="c", subcore_axis_name="s", num_cores=NC, num_subcores=NS)
_SC_PARAMS = pltpu.CompilerParams(needs_layout_passes=False)


def _zero_rows(rows):
    def body(r, _):
        for j in range(D // L):
            rows[r, pl.ds(j * L, L)] = jnp.zeros((L,), jnp.float32)
        return 0
    lax.fori_loop(0, BLK, body, 0)


def _deg_body(idx_hbm, degp_hbm, idx_all, deg_l):
    c = lax.axis_index("c")
    s = lax.axis_index("s")
    wid = s * NC + c

    pltpu.sync_copy(idx_hbm.at[pl.ds(wid * NB * 3, NB * 3)], idx_all)

    def zero(i, _):
        deg_l[pl.ds(i * L, L)] = jnp.zeros((L,), jnp.float32)
        return 0
    lax.fori_loop(0, NP // L, zero, 0)

    def blk(b, _):
        def vec(i, _):
            idx = idx_all[3 * b + 1, pl.ds(i * L, L)]
            val = plsc.bitcast(idx_all[3 * b + 2, pl.ds(i * L, L)],
                               jnp.float32)
            plsc.addupdate_scatter(deg_l, [idx], val)
            return 0
        lax.fori_loop(0, BLK // L, vec, 0)
        return 0
    lax.fori_loop(0, NB, blk, 0)

    pltpu.sync_copy(deg_l, degp_hbm.at[wid])


_deg_kernel = functools.partial(
    pl.kernel,
    out_type=jax.ShapeDtypeStruct((NW, NP), jnp.float32),
    mesh=_MESH,
    compiler_params=_SC_PARAMS,
    scratch_types=[
        pltpu.VMEM((NB * 3, BLK), jnp.int32),
        pltpu.VMEM((NP,), jnp.float32),
    ],
)(_deg_body)


HB = NB // 2


def _agg_body(idx_hbm, h_hbm, aggp_hbm,
              acc_sh, idx_all, rows0, rows1, sg0, sg1, ss0, ss1):
    c = lax.axis_index("c")
    s = lax.axis_index("s")
    wid = s * NC + c

    _zero_rows(rows0)
    for k in range(RPT // BLK):
        pltpu.sync_copy(rows0, acc_sh.at[pl.ds(s * RPT + k * BLK, BLK)])
    plsc.subcore_barrier()

    def gather(b, rows, sem):
        pltpu.async_copy(h_hbm.at[idx_all.at[3 * b]], rows, sem)

    def wait_gather(b, rows, sem):
        pltpu.make_async_copy(h_hbm.at[idx_all.at[3 * b]], rows, sem).wait()

    def scatter(b, rows, sem):
        pltpu.async_copy(rows, acc_sh.at[idx_all.at[3 * b + 1]], sem,
                         add=True)

    def wait_scatter(b, rows, sem):
        pltpu.make_async_copy(rows, acc_sh.at[idx_all.at[3 * b + 1]],
                              sem).wait()

    def scale(b, rows):
        def grp(g, _):
            wv = plsc.bitcast(idx_all[3 * b + 2, pl.ds(g * L, L)],
                              jnp.float32)
            for r in range(L):
                nrm = wv[r]
                for j in range(D // L):
                    rows[g * L + r, pl.ds(j * L, L)] = (
                        rows[g * L + r, pl.ds(j * L, L)] * nrm)
            return 0
        lax.fori_loop(0, BLK // L, grp, 0)

    for p in range(2):
        pltpu.sync_copy(
            idx_hbm.at[pl.ds((wid * NB + p * HB) * 3, HB * 3)], idx_all)
        gather(0, rows0, sg0)
        gather(1, rows1, sg1)

        def body(k, _):
            l0 = 2 * k
            l1 = l0 + 1
            wait_gather(l0, rows0, sg0)
            scale(l0, rows0)
            scatter(l0, rows0, ss0)
            wait_gather(l1, rows1, sg1)
            scale(l1, rows1)
            scatter(l1, rows1, ss1)
            wait_scatter(l0, rows0, ss0)

            @pl.when(k < HB // 2 - 1)
            def _():
                gather(l0 + 2, rows0, sg0)
            wait_scatter(l1, rows1, ss1)

            @pl.when(k < HB // 2 - 1)
            def _():
                gather(l1 + 2, rows1, sg1)
            return 0
        lax.fori_loop(0, HB // 2, body, 0)

    plsc.subcore_barrier()
    pltpu.sync_copy(acc_sh.at[pl.ds(s * RPT, RPT)],
                    aggp_hbm.at[c, pl.ds(s * RPT, RPT)])


_agg_kernel = functools.partial(
    pl.kernel,
    out_type=jax.ShapeDtypeStruct((NC, NP, D), jnp.float32),
    mesh=_MESH,
    compiler_params=_SC_PARAMS,
    scratch_types=[
        pltpu.VMEM_SHARED((NP, D), jnp.float32),
        pltpu.VMEM((HB * 3, BLK), jnp.int32),
        pltpu.VMEM((BLK, D), jnp.float32),
        pltpu.VMEM((BLK, D), jnp.float32),
        pltpu.SemaphoreType.DMA,
        pltpu.SemaphoreType.DMA,
        pltpu.SemaphoreType.DMA,
        pltpu.SemaphoreType.DMA,
    ],
)(_agg_body)


def _dinv_body(degp_ref, dinv_ref):
    deg = jnp.sum(degp_ref[...], axis=0) + 1.0
    dinv_ref[...] = jnp.where(deg > 0, lax.rsqrt(deg), 0.0)


_dinv = pl.pallas_call(
    _dinv_body,
    out_shape=jax.ShapeDtypeStruct((NP // 128, 128), jnp.float32))


def _mm1_body(x_ref, w_ref, dinv_ref, h_ref):
    h_ref[...] = dinv_ref[...] * jnp.dot(x_ref[...], w_ref[...],
                                         preferred_element_type=jnp.float32)


_mm1 = pl.pallas_call(
    _mm1_body,
    out_shape=jax.ShapeDtypeStruct((N, D), jnp.float32))


def _mid_body(aggp_ref, h_ref, dinv_ref, b_ref, w_ref, h2_ref):
    z = dinv_ref[...] * (aggp_ref[0, :N] + aggp_ref[1, :N] + h_ref[...])
    z = jnp.maximum(z + b_ref[...], 0.0)
    h2_ref[...] = dinv_ref[...] * jnp.dot(z, w_ref[...],
                                          preferred_element_type=jnp.float32)


_mid = pl.pallas_call(
    _mid_body,
    out_shape=jax.ShapeDtypeStruct((N, D), jnp.float32))


def _out_body(aggp_ref, h_ref, dinv_ref, b_ref, x_ref, out_ref):
    z = dinv_ref[...] * (aggp_ref[0, :N] + aggp_ref[1, :N] + h_ref[...])
    out_ref[:, :D] = jnp.maximum(z + b_ref[...], 0.0)
    out_ref[:, D:] = x_ref[...]


_out = pl.pallas_call(
    _out_body,
    out_shape=jax.ShapeDtypeStruct((N, 2 * D), jnp.float32))


def kernel(x, edge_index, edge_weight, W1, b1, W2, b2):
    src = edge_index[0].astype(jnp.int32)
    dst = edge_index[1].astype(jnp.int32)
    w = edge_weight.astype(jnp.float32)
    pad = EPAD - E
    src_p = jnp.pad(src, (0, pad)).reshape(NW * NB, 1, BLK)
    dst_p = jnp.pad(dst, (0, pad)).reshape(NW * NB, 1, BLK)
    wbits = lax.bitcast_convert_type(jnp.pad(w, (0, pad)),
                                     jnp.int32).reshape(NW * NB, 1, BLK)
    packed = jnp.concatenate([src_p, dst_p, wbits],
                             axis=1).reshape(NW * NB * 3, BLK)

    degp = _deg_kernel(packed)
    dinv2d = _dinv(degp.reshape(NW, NP // 128, 128))
    dinv_col = dinv2d.reshape(NP, 1)[:N]

    h1 = _mm1(x, W1, dinv_col)
    p = _agg_kernel(packed, h1)
    h2 = _mid(p, h1, dinv_col, b1.reshape(1, D), W2)
    q = _agg_kernel(packed, h2)
    return _out(q, h2, dinv_col, b2.reshape(1, D), x)


# speedup vs baseline: 9.5093x; 1.1125x over previous
import functools

import jax
import jax.numpy as jnp
from jax import lax
from jax.experimental import pallas as pl
from jax.experimental.pallas import tpu as pltpu
from jax.experimental.pallas import tpu_sc as plsc

N = 10000
D = 128
E = 320000

NC = 2
NS = 16
L = 16
NW = NC * NS

BLK = 128
NB = 80
EPT = NB * BLK
EPAD = EPT * NW
NP = 10240
FAST_CORE = 0
PHB = 40
NPH = 3
NBF = PHB * NPH
RPT = NP // NS

_MESH = plsc.VectorSubcoreMesh(
    core_axis_name="c", subcore_axis_name="s", num_cores=NC, num_subcores=NS)
_SC_PARAMS = pltpu.CompilerParams(needs_layout_passes=False)


def _zero_rows(rows):
    def body(r, _):
        for j in range(D // L):
            rows[r, pl.ds(j * L, L)] = jnp.zeros((L,), jnp.float32)
        return 0
    lax.fori_loop(0, BLK, body, 0)


def _deg_body(idx_hbm, degp_hbm, idx_all, deg_l):
    c = lax.axis_index("c")
    s = lax.axis_index("s")
    wid = s * NC + c

    pltpu.sync_copy(idx_hbm.at[pl.ds(wid * NB * 3, NB * 3)], idx_all)

    def zero(i, _):
        deg_l[pl.ds(i * L, L)] = jnp.zeros((L,), jnp.float32)
        return 0
    lax.fori_loop(0, NP // L, zero, 0)

    def blk(b, _):
        def vec(i, _):
            idx = idx_all[3 * b + 1, pl.ds(i * L, L)]
            val = plsc.bitcast(idx_all[3 * b + 2, pl.ds(i * L, L)],
                               jnp.float32)
            plsc.addupdate_scatter(deg_l, [idx], val)
            return 0
        lax.fori_loop(0, BLK // L, vec, 0)
        return 0
    lax.fori_loop(0, NB, blk, 0)

    pltpu.sync_copy(deg_l, degp_hbm.at[wid])


_deg_kernel = functools.partial(
    pl.kernel,
    out_type=jax.ShapeDtypeStruct((NW, NP), jnp.float32),
    mesh=_MESH,
    compiler_params=_SC_PARAMS,
    scratch_types=[
        pltpu.VMEM((NB * 3, BLK), jnp.int32),
        pltpu.VMEM((NP,), jnp.float32),
    ],
)(_deg_body)


HB = NB // 2


def _agg_body(idx_hbm, h_hbm, aggp_hbm,
              acc_sh, idx_all, rows0, rows1, sg0, sg1, ss0, ss1):
    c = lax.axis_index("c")
    s = lax.axis_index("s")
    wid = s * NC + c

    _zero_rows(rows0)
    for k in range(RPT // BLK):
        pltpu.sync_copy(rows0, acc_sh.at[pl.ds(s * RPT + k * BLK, BLK)])
    plsc.subcore_barrier()

    def gather(b, rows, sem):
        pltpu.async_copy(h_hbm.at[idx_all.at[3 * b]], rows, sem)

    def wait_gather(b, rows, sem):
        pltpu.make_async_copy(h_hbm.at[idx_all.at[3 * b]], rows, sem).wait()

    def scatter(b, rows, sem):
        pltpu.async_copy(rows, acc_sh.at[idx_all.at[3 * b + 1]], sem,
                         add=True)

    def wait_scatter(b, rows, sem):
        pltpu.make_async_copy(rows, acc_sh.at[idx_all.at[3 * b + 1]],
                              sem).wait()

    def scale(b, rows):
        def grp(g, _):
            wv = plsc.bitcast(idx_all[3 * b + 2, pl.ds(g * L, L)],
                              jnp.float32)
            for r in range(L):
                nrm = wv[r]
                for j in range(D // L):
                    rows[g * L + r, pl.ds(j * L, L)] = (
                        rows[g * L + r, pl.ds(j * L, L)] * nrm)
            return 0
        lax.fori_loop(0, BLK // L, grp, 0)

    heavy = c == FAST_CORE
    base_rows = jnp.where(heavy, s * (NBF * 3),
                          NS * NBF * 3 + s * (PHB * 3))

    def phase(p):
        pltpu.sync_copy(
            idx_hbm.at[pl.ds(base_rows + p * (PHB * 3), PHB * 3)], idx_all)
        gather(0, rows0, sg0)
        gather(1, rows1, sg1)

        def body(k, _):
            l0 = 2 * k
            l1 = l0 + 1
            wait_gather(l0, rows0, sg0)
            scale(l0, rows0)
            scatter(l0, rows0, ss0)
            wait_gather(l1, rows1, sg1)
            scale(l1, rows1)
            scatter(l1, rows1, ss1)
            wait_scatter(l0, rows0, ss0)

            @pl.when(k < PHB // 2 - 1)
            def _():
                gather(l0 + 2, rows0, sg0)
            wait_scatter(l1, rows1, ss1)

            @pl.when(k < PHB // 2 - 1)
            def _():
                gather(l1 + 2, rows1, sg1)
            return 0
        lax.fori_loop(0, PHB // 2, body, 0)

    phase(0)
    for p in range(1, NPH):
        pl.when(heavy)(functools.partial(phase, p))

    plsc.subcore_barrier()
    pltpu.sync_copy(acc_sh.at[pl.ds(s * RPT, RPT)],
                    aggp_hbm.at[c, pl.ds(s * RPT, RPT)])


_agg_kernel = functools.partial(
    pl.kernel,
    out_type=jax.ShapeDtypeStruct((NC, NP, D), jnp.float32),
    mesh=_MESH,
    compiler_params=_SC_PARAMS,
    scratch_types=[
        pltpu.VMEM_SHARED((NP, D), jnp.float32),
        pltpu.VMEM((PHB * 3, BLK), jnp.int32),
        pltpu.VMEM((BLK, D), jnp.float32),
        pltpu.VMEM((BLK, D), jnp.float32),
        pltpu.SemaphoreType.DMA,
        pltpu.SemaphoreType.DMA,
        pltpu.SemaphoreType.DMA,
        pltpu.SemaphoreType.DMA,
    ],
)(_agg_body)


def _dinv_body(degp_ref, dinv_ref):
    deg = jnp.sum(degp_ref[...], axis=0) + 1.0
    dinv_ref[...] = jnp.where(deg > 0, lax.rsqrt(deg), 0.0)


_dinv = pl.pallas_call(
    _dinv_body,
    out_shape=jax.ShapeDtypeStruct((NP // 128, 128), jnp.float32))


def _mm1_body(x_ref, w_ref, dinv_ref, h_ref):
    h_ref[...] = dinv_ref[...] * jnp.dot(x_ref[...], w_ref[...],
                                         preferred_element_type=jnp.float32)


_mm1 = pl.pallas_call(
    _mm1_body,
    out_shape=jax.ShapeDtypeStruct((N, D), jnp.float32))


def _mid_body(aggp_ref, h_ref, dinv_ref, b_ref, w_ref, h2_ref):
    z = dinv_ref[...] * (aggp_ref[0, :N] + aggp_ref[1, :N] + h_ref[...])
    z = jnp.maximum(z + b_ref[...], 0.0)
    h2_ref[...] = dinv_ref[...] * jnp.dot(z, w_ref[...],
                                          preferred_element_type=jnp.float32)


_mid = pl.pallas_call(
    _mid_body,
    out_shape=jax.ShapeDtypeStruct((N, D), jnp.float32))


def _out_body(aggp_ref, h_ref, dinv_ref, b_ref, x_ref, out_ref):
    z = dinv_ref[...] * (aggp_ref[0, :N] + aggp_ref[1, :N] + h_ref[...])
    out_ref[:, :D] = jnp.maximum(z + b_ref[...], 0.0)
    out_ref[:, D:] = x_ref[...]


_out = pl.pallas_call(
    _out_body,
    out_shape=jax.ShapeDtypeStruct((N, 2 * D), jnp.float32))


def kernel(x, edge_index, edge_weight, W1, b1, W2, b2):
    src = edge_index[0].astype(jnp.int32)
    dst = edge_index[1].astype(jnp.int32)
    w = edge_weight.astype(jnp.float32)
    pad = EPAD - E
    src_p = jnp.pad(src, (0, pad)).reshape(NW * NB, 1, BLK)
    dst_p = jnp.pad(dst, (0, pad)).reshape(NW * NB, 1, BLK)
    wbits = lax.bitcast_convert_type(jnp.pad(w, (0, pad)),
                                     jnp.int32).reshape(NW * NB, 1, BLK)
    packed = jnp.concatenate([src_p, dst_p, wbits],
                             axis=1).reshape(NW * NB * 3, BLK)

    degp = _deg_kernel(packed)
    dinv2d = _dinv(degp.reshape(NW, NP // 128, 128))
    dinv_col = dinv2d.reshape(NP, 1)[:N]

    h1 = _mm1(x, W1, dinv_col)
    p = _agg_kernel(packed, h1)
    h2 = _mid(p, h1, dinv_col, b1.reshape(1, D), W2)
    q = _agg_kernel(packed, h2)
    return _out(q, h2, dinv_col, b2.reshape(1, D), x)
